# baseline (speedup 1.0000x reference)
import functools

import jax
import jax.numpy as jnp
from jax.experimental import pallas as pl

_TOP_K = 8
_NEG_INF = float("-inf")


def _gate_body(x_ref, w_ref, idx_ref, wgt_ref):
    x = x_ref[:]
    w = w_ref[:]
    logits = jax.lax.dot_general(
        x, w, (((1,), (1,)), ((), ())), preferred_element_type=jnp.float32
    )

    bt, e = logits.shape
    lane = jax.lax.broadcasted_iota(jnp.int32, (bt, e), 1)

    bits = jax.lax.bitcast_convert_type(logits, jnp.int32)
    sign = jax.lax.shift_right_arithmetic(bits, 31)
    tag = jnp.bitwise_xor(jnp.int32(e - 1) - lane, jnp.bitwise_and(sign, 0x3F))
    kbits = jnp.bitwise_or(jnp.bitwise_and(bits, jnp.int32(~0x3F)), tag)
    key = jax.lax.bitcast_convert_type(kbits, jnp.float32)

    keys = []
    cur = key
    for _ in range(_TOP_K):
        m = jnp.max(cur, axis=-1, keepdims=True)
        keys.append(m)
        cur = jnp.where(cur == m, _NEG_INF, cur)

    topk = jnp.concatenate(keys, axis=-1)
    tbits = jax.lax.bitcast_convert_type(topk, jnp.int32)
    tsign = jax.lax.shift_right_arithmetic(tbits, 31)
    ttag = jnp.bitwise_xor(
        jnp.bitwise_and(tbits, jnp.int32(0x3F)), jnp.bitwise_and(tsign, 0x3F)
    )
    topi = jnp.int32(e - 1) - ttag

    topv = jax.lax.bitcast_convert_type(
        jnp.bitwise_and(tbits, jnp.int32(~0x3F)), jnp.float32
    )

    ex = jnp.exp(topv - topv[:, 0:1])
    wgt = ex / jnp.sum(ex, axis=-1, keepdims=True)

    idx_ref[:] = topi
    wgt_ref[:] = wgt


@functools.partial(jax.jit, static_argnames=())
def _gate(flat, weight):
    t, h = flat.shape
    e = weight.shape[0]
    bt = 1024
    grid = (t // bt,)
    topi, topw = pl.pallas_call(
        _gate_body,
        grid=grid,
        in_specs=[
            pl.BlockSpec((bt, h), lambda i: (i, 0)),
            pl.BlockSpec((e, h), lambda i: (0, 0)),
        ],
        out_specs=[
            pl.BlockSpec((bt, _TOP_K), lambda i: (i, 0)),
            pl.BlockSpec((bt, _TOP_K), lambda i: (i, 0)),
        ],
        out_shape=[
            jax.ShapeDtypeStruct((t, _TOP_K), jnp.int32),
            jax.ShapeDtypeStruct((t, _TOP_K), jnp.float32),
        ],
        compiler_params=pltpu_params(),
    )(flat, weight)
    return topi, topw


def pltpu_params():
    from jax.experimental.pallas import tpu as pltpu

    return pltpu.CompilerParams(dimension_semantics=("parallel",))


def kernel(hidden_states, weight):
    bsz, seq_len, h = hidden_states.shape
    flat = hidden_states.reshape(-1, h)
    topi, topw = _gate(flat, weight)
    aux_loss = jnp.float32(0.0)
    return (topi, topw, aux_loss)


# speedup vs baseline: 1.0924x; 1.0924x over previous
import functools

import jax
import jax.numpy as jnp
from jax.experimental import pallas as pl

_TOP_K = 8
_NEG_INF = float("-inf")


_N_SUB = 4


def _gate_body(x_ref, w_ref, idx_ref, wgt_ref):
    w = w_ref[:]
    sb = x_ref.shape[0] // _N_SUB
    for s in range(_N_SUB):
        x = x_ref[pl.ds(s * sb, sb), :]
        topi, wgt = _topk_softmax(
            jax.lax.dot_general(
                x, w, (((1,), (1,)), ((), ())), preferred_element_type=jnp.float32
            )
        )
        idx_ref[pl.ds(s * sb, sb), :] = topi
        wgt_ref[pl.ds(s * sb, sb), :] = wgt


def _topk_softmax(logits):
    bt, e = logits.shape
    lane = jax.lax.broadcasted_iota(jnp.int32, (bt, e), 1)

    bits = jax.lax.bitcast_convert_type(logits, jnp.int32)
    sign = jax.lax.shift_right_arithmetic(bits, 31)
    tag = jnp.bitwise_xor(jnp.int32(e - 1) - lane, jnp.bitwise_and(sign, 0x3F))
    kbits = jnp.bitwise_or(jnp.bitwise_and(bits, jnp.int32(~0x3F)), tag)
    key = jax.lax.bitcast_convert_type(kbits, jnp.float32)

    keys = []
    cur = key
    for _ in range(_TOP_K):
        m = jnp.max(cur, axis=-1, keepdims=True)
        keys.append(m)
        cur = jnp.where(cur == m, _NEG_INF, cur)

    topk = jnp.concatenate(keys, axis=-1)
    tbits = jax.lax.bitcast_convert_type(topk, jnp.int32)
    tsign = jax.lax.shift_right_arithmetic(tbits, 31)
    ttag = jnp.bitwise_xor(
        jnp.bitwise_and(tbits, jnp.int32(0x3F)), jnp.bitwise_and(tsign, 0x3F)
    )
    topi = jnp.int32(e - 1) - ttag

    topv = jax.lax.bitcast_convert_type(
        jnp.bitwise_and(tbits, jnp.int32(~0x3F)), jnp.float32
    )

    ex = jnp.exp(topv - topv[:, 0:1])
    wgt = ex / jnp.sum(ex, axis=-1, keepdims=True)
    return topi, wgt


@functools.partial(jax.jit, static_argnames=())
def _gate(flat, weight):
    t, h = flat.shape
    e = weight.shape[0]
    bt = 1024
    grid = (t // bt,)
    topi, topw = pl.pallas_call(
        _gate_body,
        grid=grid,
        in_specs=[
            pl.BlockSpec((bt, h), lambda i: (i, 0)),
            pl.BlockSpec((e, h), lambda i: (0, 0)),
        ],
        out_specs=[
            pl.BlockSpec((bt, _TOP_K), lambda i: (i, 0)),
            pl.BlockSpec((bt, _TOP_K), lambda i: (i, 0)),
        ],
        out_shape=[
            jax.ShapeDtypeStruct((t, _TOP_K), jnp.int32),
            jax.ShapeDtypeStruct((t, _TOP_K), jnp.float32),
        ],
        compiler_params=pltpu_params(),
    )(flat, weight)
    return topi, topw


def pltpu_params():
    from jax.experimental.pallas import tpu as pltpu

    return pltpu.CompilerParams(dimension_semantics=("parallel",))


def kernel(hidden_states, weight):
    bsz, seq_len, h = hidden_states.shape
    flat = hidden_states.reshape(-1, h)
    topi, topw = _gate(flat, weight)
    aux_loss = jnp.float32(0.0)
    return (topi, topw, aux_loss)


# speedup vs baseline: 1.0946x; 1.0020x over previous
import functools

import jax
import jax.numpy as jnp
from jax.experimental import pallas as pl

_TOP_K = 8
_NEG_INF = float("-inf")


_N_SUB = 4


def _gate_body(x1_ref, x2_ref, w_ref, idx_ref, wgt_ref):
    w = w_ref[:]
    hh = x1_ref.shape[1]
    sb = x1_ref.shape[0] // _N_SUB
    for s in range(_N_SUB):
        rows = pl.ds(s * sb, sb)
        dn = (((1,), (1,)), ((), ()))
        logits = jax.lax.dot_general(
            x1_ref[rows, :], w[:, :hh], dn, preferred_element_type=jnp.float32
        ) + jax.lax.dot_general(
            x2_ref[rows, :], w[:, hh:], dn, preferred_element_type=jnp.float32
        )
        topi, wgt = _topk_softmax(logits)
        idx_ref[rows, :] = topi
        wgt_ref[rows, :] = wgt


def _topk_softmax(logits):
    bt, e = logits.shape
    lane = jax.lax.broadcasted_iota(jnp.int32, (bt, e), 1)

    bits = jax.lax.bitcast_convert_type(logits, jnp.int32)
    sign = jax.lax.shift_right_arithmetic(bits, 31)
    tag = jnp.bitwise_xor(jnp.int32(e - 1) - lane, jnp.bitwise_and(sign, 0x3F))
    kbits = jnp.bitwise_or(jnp.bitwise_and(bits, jnp.int32(~0x3F)), tag)
    key = jax.lax.bitcast_convert_type(kbits, jnp.float32)

    keys = []
    cur = key
    for _ in range(_TOP_K):
        m = jnp.max(cur, axis=-1, keepdims=True)
        keys.append(m)
        cur = jnp.where(cur == m, _NEG_INF, cur)

    topk = jnp.concatenate(keys, axis=-1)
    tbits = jax.lax.bitcast_convert_type(topk, jnp.int32)
    tsign = jax.lax.shift_right_arithmetic(tbits, 31)
    ttag = jnp.bitwise_xor(
        jnp.bitwise_and(tbits, jnp.int32(0x3F)), jnp.bitwise_and(tsign, 0x3F)
    )
    topi = jnp.int32(e - 1) - ttag

    topv = jax.lax.bitcast_convert_type(
        jnp.bitwise_and(tbits, jnp.int32(~0x3F)), jnp.float32
    )

    ex = jnp.exp(topv - topv[:, 0:1])
    wgt = ex / jnp.sum(ex, axis=-1, keepdims=True)
    return topi, wgt


@functools.partial(jax.jit, static_argnames=())
def _gate(flat, weight):
    t, h = flat.shape
    e = weight.shape[0]
    bt = 1024
    grid = (t // bt,)
    topi, topw = pl.pallas_call(
        _gate_body,
        grid=grid,
        in_specs=[
            pl.BlockSpec((bt, h // 2), lambda i: (i, 0)),
            pl.BlockSpec((bt, h // 2), lambda i: (i, 1)),
            pl.BlockSpec((e, h), lambda i: (0, 0)),
        ],
        out_specs=[
            pl.BlockSpec((bt, _TOP_K), lambda i: (i, 0)),
            pl.BlockSpec((bt, _TOP_K), lambda i: (i, 0)),
        ],
        out_shape=[
            jax.ShapeDtypeStruct((t, _TOP_K), jnp.int32),
            jax.ShapeDtypeStruct((t, _TOP_K), jnp.float32),
        ],
        compiler_params=pltpu_params(),
    )(flat, flat, weight)
    return topi, topw


def pltpu_params():
    from jax.experimental.pallas import tpu as pltpu

    return pltpu.CompilerParams(dimension_semantics=("parallel",))


def kernel(hidden_states, weight):
    bsz, seq_len, h = hidden_states.shape
    flat = hidden_states.reshape(-1, h)
    topi, topw = _gate(flat, weight)
    aux_loss = jnp.float32(0.0)
    return (topi, topw, aux_loss)
